# Initial kernel scaffold; baseline (speedup 1.0000x reference)
import jax
import jax.numpy as jnp
from jax.experimental import pallas as pl


def kernel(enc_inputs, src_emb, pos_emb):
    raise NotImplementedError("write your pallas kernel here")


# speedup vs baseline: 3.6573x; 3.6573x over previous
import functools

import jax
import jax.numpy as jnp
from jax import lax
from jax.experimental import pallas as pl
from jax.experimental.pallas import tpu as pltpu
from jax.experimental.pallas import tpu_sc as plsc

CHUNK = 128
NBUF = 2


@functools.lru_cache(maxsize=None)
def _make_kernel(B, L, D, V):
    mesh = plsc.VectorSubcoreMesh(core_axis_name="c", subcore_axis_name="s")
    NC, NS = mesh.num_cores, mesh.num_subcores
    NW = NC * NS
    assert B == NW * CHUNK
    assert L % NBUF == 0
    assert D % 16 == 0 and D <= 128

    @functools.partial(
        pl.kernel,
        out_type=jax.ShapeDtypeStruct((L, D, B), jnp.float32),
        mesh=mesh,
        compiler_params=pltpu.CompilerParams(needs_layout_passes=False),
        scratch_types=[
            pltpu.VMEM((L, CHUNK), jnp.int32),
            pltpu.VMEM((L, 128), jnp.float32),
            pltpu.VMEM((NBUF, CHUNK, 128), jnp.float32),
            pltpu.VMEM((NBUF, D, CHUNK), jnp.float32),
            [pltpu.SemaphoreType.DMA] * NBUF,
            [pltpu.SemaphoreType.DMA] * NBUF,
        ],
    )
    def enc_kernel(idx_hbm, src_hbm, pos_hbm, out_hbm, idx_v, pos_v, g_ring,
                   t_ring, gsems, ssems):
        cid = lax.axis_index("c")
        sid = lax.axis_index("s")
        wid = sid * NC + cid
        b0 = wid * CHUNK

        pltpu.sync_copy(idx_hbm.at[:, pl.ds(b0, CHUNK)], idx_v)
        pltpu.sync_copy(pos_hbm, pos_v)

        iota16 = lax.iota(jnp.int32, 16)
        zero16 = iota16 * 0

        def start_gather(l, b):
            pltpu.async_copy(src_hbm.at[idx_v.at[l]], g_ring.at[b], gsems[b])

        def wait_gather(b):
            pltpu.make_async_copy(
                src_hbm.at[idx_v.at[0]], g_ring.at[b], gsems[b]).wait()

        def transpose_add(l, b):
            @pl.loop(0, D, unroll=2)
            def _(d):
                lidx = zero16 + l
                cidx = zero16 + d
                pv = plsc.load_gather(pos_v, [lidx, cidx])
                for j in range(CHUNK // 16):
                    ridx = iota16 + (j * 16)
                    vals = plsc.load_gather(g_ring.at[b], [ridx, cidx])
                    t_ring[b, d, pl.ds(j * 16, 16)] = vals + pv

        def start_store(l, b):
            pltpu.async_copy(t_ring.at[b],
                             out_hbm.at[l, :, pl.ds(b0, CHUNK)], ssems[b])

        def wait_store(b):
            pltpu.make_async_copy(
                t_ring.at[b], out_hbm.at[0, :, pl.ds(0, CHUNK)],
                ssems[b]).wait()

        for b in range(NBUF):
            start_gather(b, b)

        @pl.loop(1, L // NBUF)
        def _(g):
            l_new0 = g * NBUF
            l_old0 = l_new0 - NBUF
            for b in range(NBUF):
                wait_gather(b)
                transpose_add(l_old0 + b, b)
                start_store(l_old0 + b, b)
            for b in range(NBUF):
                wait_store(b)
                start_gather(l_new0 + b, b)

        for b in range(NBUF):
            wait_gather(b)
            transpose_add(L - NBUF + b, b)
            start_store(L - NBUF + b, b)
        for b in range(NBUF):
            wait_store(b)

    return enc_kernel


def kernel(enc_inputs, src_emb, pos_emb):
    B, L = enc_inputs.shape
    V, D = src_emb.shape
    idx_t = enc_inputs.T
    src_pad = jnp.pad(src_emb, ((0, 0), (0, 128 - D)))
    pos_pad = jnp.pad(pos_emb[:L], ((0, 0), (0, 128 - D)))
    out3 = _make_kernel(B, L, D, V)(idx_t, src_pad, pos_pad)
    return jnp.transpose(out3, (2, 0, 1))


# speedup vs baseline: 7.7926x; 2.1307x over previous
import functools

import jax
import jax.numpy as jnp
from jax import lax
from jax.experimental import pallas as pl
from jax.experimental.pallas import tpu as pltpu
from jax.experimental.pallas import tpu_sc as plsc

CHUNK = 128
NBUF = 2


@functools.lru_cache(maxsize=None)
def _make_kernel(B, L, D, V):
    mesh = plsc.VectorSubcoreMesh(core_axis_name="c", subcore_axis_name="s")
    NC, NS = mesh.num_cores, mesh.num_subcores
    NW = NC * NS
    assert B == NW * CHUNK
    assert L % NBUF == 0
    assert D % 16 == 0 and D <= 128

    @functools.partial(
        pl.kernel,
        out_type=jax.ShapeDtypeStruct((L, D, B), jnp.float32),
        mesh=mesh,
        compiler_params=pltpu.CompilerParams(needs_layout_passes=False),
        scratch_types=[
            pltpu.VMEM((L, CHUNK), jnp.int32),
            pltpu.VMEM((L, 128), jnp.float32),
            pltpu.VMEM((NBUF, CHUNK, 128), jnp.float32),
            pltpu.VMEM((NBUF, D, CHUNK), jnp.float32),
            [pltpu.SemaphoreType.DMA] * NBUF,
            [pltpu.SemaphoreType.DMA] * NBUF,
        ],
    )
    def enc_kernel(idx_hbm, src_hbm, pos_hbm, out_hbm, idx_v, pos_v, g_ring,
                   t_ring, gsems, ssems):
        cid = lax.axis_index("c")
        sid = lax.axis_index("s")
        wid = sid * NC + cid
        b0 = wid * CHUNK

        pltpu.sync_copy(idx_hbm.at[:, pl.ds(b0, CHUNK)], idx_v)
        pltpu.sync_copy(pos_hbm, pos_v)

        iota16 = lax.iota(jnp.int32, 16)
        zero16 = iota16 * 0

        def start_gather(l, b):
            pltpu.async_copy(src_hbm.at[idx_v.at[l]], g_ring.at[b], gsems[b])

        def wait_gather(b):
            pltpu.make_async_copy(
                src_hbm.at[idx_v.at[0]], g_ring.at[b], gsems[b]).wait()

        rows_const = [iota16 + (j * 16) for j in range(CHUNK // 16)]

        def transpose_add(l, b):
            @pl.loop(0, 16, unroll=2)
            def _(k):
                lsplat = zero16 + l
                rot = (iota16 + k) & 15
                for dc in range(D // 16):
                    gcol = rot + (dc * 16)
                    pv = plsc.load_gather(pos_v, [lsplat, gcol])
                    for j in range(CHUNK // 16):
                        v = plsc.load_gather(g_ring.at[b],
                                             [rows_const[j], gcol])
                        plsc.store_scatter(t_ring.at[b],
                                           [gcol, rows_const[j]], v + pv)

        def start_store(l, b):
            pltpu.async_copy(t_ring.at[b],
                             out_hbm.at[l, :, pl.ds(b0, CHUNK)], ssems[b])

        def wait_store(b):
            pltpu.make_async_copy(
                t_ring.at[b], out_hbm.at[0, :, pl.ds(0, CHUNK)],
                ssems[b]).wait()

        for b in range(NBUF):
            start_gather(b, b)

        @pl.loop(1, L // NBUF)
        def _(g):
            l_new0 = g * NBUF
            l_old0 = l_new0 - NBUF
            for b in range(NBUF):
                wait_gather(b)
                transpose_add(l_old0 + b, b)
                start_store(l_old0 + b, b)
            for b in range(NBUF):
                wait_store(b)
                start_gather(l_new0 + b, b)

        for b in range(NBUF):
            wait_gather(b)
            transpose_add(L - NBUF + b, b)
            start_store(L - NBUF + b, b)
        for b in range(NBUF):
            wait_store(b)

    return enc_kernel


def kernel(enc_inputs, src_emb, pos_emb):
    B, L = enc_inputs.shape
    V, D = src_emb.shape
    idx_t = enc_inputs.T
    src_pad = jnp.pad(src_emb, ((0, 0), (0, 128 - D)))
    pos_pad = jnp.pad(pos_emb[:L], ((0, 0), (0, 128 - D)))
    out3 = _make_kernel(B, L, D, V)(idx_t, src_pad, pos_pad)
    return jnp.transpose(out3, (2, 0, 1))


# speedup vs baseline: 9.9108x; 1.2718x over previous
import functools

import jax
import jax.numpy as jnp
from jax import lax
from jax.experimental import pallas as pl
from jax.experimental.pallas import tpu as pltpu
from jax.experimental.pallas import tpu_sc as plsc

CHUNK = 128
NBUF_G = 4
NBUF_T = 2


@functools.lru_cache(maxsize=None)
def _make_kernel(B, L, D, V):
    mesh = plsc.VectorSubcoreMesh(core_axis_name="c", subcore_axis_name="s")
    NC, NS = mesh.num_cores, mesh.num_subcores
    NW = NC * NS
    assert B == NW * CHUNK
    assert L % NBUF_G == 0 and L % 2 == 0
    assert D % 16 == 0 and D <= 128

    @functools.partial(
        pl.kernel,
        out_type=jax.ShapeDtypeStruct((L, D, B), jnp.float32),
        mesh=mesh,
        compiler_params=pltpu.CompilerParams(needs_layout_passes=False),
        scratch_types=[
            pltpu.VMEM((L, CHUNK), jnp.int32),
            pltpu.VMEM((L // 2, 128), jnp.float32),
            pltpu.VMEM((NBUF_G, CHUNK, 128), jnp.float32),
            pltpu.VMEM((NBUF_T, D, CHUNK), jnp.float32),
            [pltpu.SemaphoreType.DMA] * NBUF_G,
            [pltpu.SemaphoreType.DMA] * NBUF_T,
        ],
    )
    def enc_kernel(idx_hbm, src_hbm, pos_hbm, out_hbm, idx_v, pos_v, g_ring,
                   t_ring, gsems, ssems):
        cid = lax.axis_index("c")
        sid = lax.axis_index("s")
        wid = sid * NC + cid
        b0 = wid * CHUNK

        pltpu.sync_copy(idx_hbm.at[:, pl.ds(b0, CHUNK)], idx_v)
        pltpu.sync_copy(pos_hbm, pos_v)

        iota16 = lax.iota(jnp.int32, 16)
        zero16 = iota16 * 0
        rows_const = [iota16 + (j * 16) for j in range(CHUNK // 16)]

        def start_gather(l, gb):
            pltpu.async_copy(src_hbm.at[idx_v.at[l]], g_ring.at[gb], gsems[gb])

        def wait_gather(gb):
            pltpu.make_async_copy(
                src_hbm.at[idx_v.at[0]], g_ring.at[gb], gsems[gb]).wait()

        def transpose_add(l, gb, tb):
            poff = (lax.rem(l, 2)) * D
            prow = lax.div(l, 2)

            @pl.loop(0, 16, unroll=2)
            def _(k):
                psplat = zero16 + prow
                rot = (iota16 + k) & 15
                for dc in range(D // 16):
                    gcol = rot + (dc * 16)
                    pv = plsc.load_gather(pos_v, [psplat, gcol + poff])
                    for j in range(CHUNK // 16):
                        v = plsc.load_gather(g_ring.at[gb],
                                             [rows_const[j], gcol])
                        plsc.store_scatter(t_ring.at[tb],
                                           [gcol, rows_const[j]], v + pv)

        def start_store(l, tb):
            pltpu.async_copy(t_ring.at[tb],
                             out_hbm.at[l, :, pl.ds(b0, CHUNK)], ssems[tb])

        def wait_store(tb):
            pltpu.make_async_copy(
                t_ring.at[tb], out_hbm.at[0, :, pl.ds(0, CHUNK)],
                ssems[tb]).wait()

        for j in range(NBUF_G):
            start_gather(j, j)

        @pl.loop(0, L // NBUF_G)
        def _(g):
            l0 = g * NBUF_G
            for j in range(NBUF_G):
                l = l0 + j
                tb = j % NBUF_T
                wait_gather(j)

                @pl.when(l >= NBUF_T)
                def _():
                    wait_store(tb)

                transpose_add(l, j, tb)
                start_store(l, tb)

                @pl.when(l + NBUF_G < L)
                def _():
                    start_gather(l + NBUF_G, j)

        for tb in range(NBUF_T):
            wait_store(tb)

    return enc_kernel


def kernel(enc_inputs, src_emb, pos_emb):
    B, L = enc_inputs.shape
    V, D = src_emb.shape
    idx_t = enc_inputs.T
    src_pad = jnp.pad(src_emb, ((0, 0), (0, 128 - D)))
    pos_packed = pos_emb[:L].reshape(L // 2, 2 * D)
    out3 = _make_kernel(B, L, D, V)(idx_t, src_pad, pos_packed)
    return jnp.transpose(out3, (2, 0, 1))
